# Initial kernel scaffold; baseline (speedup 1.0000x reference)
import jax
import jax.numpy as jnp
from jax.experimental import pallas as pl


def kernel(x, edge_index, W1, b1, W2, b2):
    raise NotImplementedError("write your pallas kernel here")


# speedup vs baseline: 34.3103x; 34.3103x over previous
import functools

import jax
import jax.numpy as jnp
from jax import lax
from jax.experimental import pallas as pl
from jax.experimental.pallas import tpu as pltpu
from jax.experimental.pallas import tpu_sc as plsc

NC = 2
NS = 16
NW = NC * NS

N = 10000
E = 320000
D = 128
H = 32

CHUNK = 128
K = 79
T = K * CHUNK
EP = NW * T
NP = 10240

_mesh = plsc.VectorSubcoreMesh(core_axis_name="c", subcore_axis_name="s")


@functools.partial(
    pl.kernel,
    out_type=jax.ShapeDtypeStruct((NC, NP), jnp.float32),
    mesh=_mesh,
    compiler_params=pltpu.CompilerParams(use_tc_tiling_on_sc=False),
    scratch_types=[
        pltpu.VMEM((K, CHUNK), jnp.int32),
        pltpu.VMEM((CHUNK,), jnp.float32),
        pltpu.VMEM_SHARED((NP,), jnp.float32),
    ],
)
def _sc_deg(dst_hbm, ones_hbm, zeros_hbm, out_hbm, dst_v, ones_v, acc_sh):
    c = lax.axis_index("c")
    s = lax.axis_index("s")
    wid = c * NS + s
    pltpu.sync_copy(dst_hbm.at[wid], dst_v)
    pltpu.sync_copy(ones_hbm, ones_v)

    @pl.when(s == 0)
    def _():
        pltpu.sync_copy(zeros_hbm, acc_sh)

    plsc.subcore_barrier()

    def step(j, _):
        pltpu.sync_copy(ones_v, acc_sh.at[dst_v.at[j]], add=True)
        return ()

    lax.fori_loop(0, K, step, ())
    plsc.subcore_barrier()

    rows_per = NP // NS
    pltpu.sync_copy(acc_sh.at[pl.ds(s * rows_per, rows_per)],
                    out_hbm.at[c].at[pl.ds(s * rows_per, rows_per)])


@functools.partial(
    pl.kernel,
    out_type=jax.ShapeDtypeStruct((NC, NP, H), jnp.float32),
    mesh=_mesh,
    compiler_params=pltpu.CompilerParams(use_tc_tiling_on_sc=False),
    scratch_types=[
        pltpu.VMEM((K, CHUNK), jnp.int32),
        pltpu.VMEM((K, CHUNK), jnp.int32),
        pltpu.VMEM((CHUNK, H), jnp.float32),
        pltpu.VMEM_SHARED((NP, H), jnp.float32),
        pltpu.SemaphoreType.DMA,
    ],
)
def _sc_agg1(src_hbm, dst_hbm, p1_hbm, zeros_hbm, out_hbm,
             src_v, dst_v, rows_v, acc_sh, sem):
    c = lax.axis_index("c")
    s = lax.axis_index("s")
    wid = c * NS + s
    pltpu.sync_copy(src_hbm.at[wid], src_v)
    pltpu.sync_copy(dst_hbm.at[wid], dst_v)

    @pl.when(s == 0)
    def _():
        pltpu.sync_copy(zeros_hbm, acc_sh)

    plsc.subcore_barrier()

    def step(j, _):
        pltpu.async_copy(p1_hbm.at[src_v.at[j]], rows_v, sem).wait()
        pltpu.sync_copy(rows_v, acc_sh.at[dst_v.at[j]], add=True)
        return ()

    lax.fori_loop(0, K, step, ())
    plsc.subcore_barrier()

    rows_per = NP // NS
    pltpu.sync_copy(acc_sh.at[pl.ds(s * rows_per, rows_per)],
                    out_hbm.at[c].at[pl.ds(s * rows_per, rows_per)])


@functools.partial(
    pl.kernel,
    out_type=jax.ShapeDtypeStruct((NC, NP), jnp.float32),
    mesh=_mesh,
    compiler_params=pltpu.CompilerParams(use_tc_tiling_on_sc=False),
    scratch_types=[
        pltpu.VMEM((K, CHUNK), jnp.int32),
        pltpu.VMEM((K, CHUNK), jnp.int32),
        pltpu.VMEM((CHUNK,), jnp.float32),
        pltpu.VMEM_SHARED((NP,), jnp.float32),
        pltpu.SemaphoreType.DMA,
    ],
)
def _sc_agg2(src_hbm, dst_hbm, p2_hbm, zeros_hbm, out_hbm,
             src_v, dst_v, rows_v, acc_sh, sem):
    c = lax.axis_index("c")
    s = lax.axis_index("s")
    wid = c * NS + s
    pltpu.sync_copy(src_hbm.at[wid], src_v)
    pltpu.sync_copy(dst_hbm.at[wid], dst_v)

    @pl.when(s == 0)
    def _():
        pltpu.sync_copy(zeros_hbm, acc_sh)

    plsc.subcore_barrier()

    def step(j, _):
        pltpu.async_copy(p2_hbm.at[src_v.at[j]], rows_v, sem).wait()
        pltpu.sync_copy(rows_v, acc_sh.at[dst_v.at[j]], add=True)
        return ()

    lax.fori_loop(0, K, step, ())
    plsc.subcore_barrier()

    rows_per = NP // NS
    pltpu.sync_copy(acc_sh.at[pl.ds(s * rows_per, rows_per)],
                    out_hbm.at[c].at[pl.ds(s * rows_per, rows_per)])


def _tc_a_body(deg_ref, x_ref, w1_ref, p1_ref, d_ref):
    deg = jnp.sum(deg_ref[...], axis=1, keepdims=True) + 1.0
    d = lax.rsqrt(deg)
    d_ref[...] = d
    p1_ref[...] = jnp.dot(x_ref[...], w1_ref[...],
                          preferred_element_type=jnp.float32) * d


def _tc_b_body(s1_ref, p1_ref, d_ref, b1_ref, w2_ref, p2_ref):
    s1 = s1_ref[0] + s1_ref[1] + p1_ref[...]
    h = jnp.maximum(s1 * d_ref[...] + b1_ref[...], 0.0)
    p2_ref[...] = jnp.dot(h, w2_ref[...],
                          preferred_element_type=jnp.float32) * d_ref[...]


def _tc_c_body(s2_ref, p2_ref, d_ref, b2_ref, out_ref):
    s2 = jnp.sum(s2_ref[...], axis=1, keepdims=True) + p2_ref[...]
    out_ref[...] = s2 * d_ref[...] + b2_ref[...]


def kernel(x, edge_index, W1, b1, W2, b2):
    src = edge_index[0]
    dst = edge_index[1]
    pad = EP - E
    padv = jnp.full((pad,), N, jnp.int32)
    src_p = jnp.concatenate([src, padv]).reshape(NW, K, CHUNK)
    dst_p = jnp.concatenate([dst, padv]).reshape(NW, K, CHUNK)
    x_p = jnp.pad(x, ((0, NP - N), (0, 0)))

    degT = _sc_deg(dst_p, jnp.ones((CHUNK,), jnp.float32),
                   jnp.zeros((NP,), jnp.float32))

    p1, d = pl.pallas_call(
        _tc_a_body,
        out_shape=(jax.ShapeDtypeStruct((NP, H), jnp.float32),
                   jax.ShapeDtypeStruct((NP, 1), jnp.float32)),
    )(degT.T, x_p, W1)

    s1p = _sc_agg1(src_p, dst_p, p1, jnp.zeros((NP, H), jnp.float32))

    p2col = pl.pallas_call(
        _tc_b_body,
        out_shape=jax.ShapeDtypeStruct((NP, 1), jnp.float32),
    )(s1p, p1, d, b1.reshape(1, H), W2)

    s2T = _sc_agg2(src_p, dst_p, p2col.reshape(NP),
                   jnp.zeros((NP,), jnp.float32))

    out = pl.pallas_call(
        _tc_c_body,
        out_shape=jax.ShapeDtypeStruct((NP, 1), jnp.float32),
    )(s2T.T, p2col, d, b2.reshape(1, 1))

    return out[:N, 0]


# speedup vs baseline: 37.0575x; 1.0801x over previous
import functools

import jax
import jax.numpy as jnp
from jax import lax
from jax.experimental import pallas as pl
from jax.experimental.pallas import tpu as pltpu
from jax.experimental.pallas import tpu_sc as plsc

NC = 2
NS = 16
NW = NC * NS

N = 10000
E = 320000
D = 128
H = 32

CHUNK = 128
K = 80
T = K * CHUNK
EP = NW * T
NP = 10240
NB = 4

_mesh = plsc.VectorSubcoreMesh(core_axis_name="c", subcore_axis_name="s")


@functools.partial(
    pl.kernel,
    out_type=jax.ShapeDtypeStruct((NC, NP), jnp.float32),
    mesh=_mesh,
    compiler_params=pltpu.CompilerParams(use_tc_tiling_on_sc=False),
    scratch_types=[
        pltpu.VMEM((K, CHUNK), jnp.int32),
        pltpu.VMEM((CHUNK,), jnp.float32),
        pltpu.VMEM_SHARED((NP,), jnp.float32),
    ],
)
def _sc_deg(dst_hbm, ones_hbm, zeros_hbm, out_hbm, dst_v, ones_v, acc_sh):
    c = lax.axis_index("c")
    s = lax.axis_index("s")
    wid = c * NS + s
    pltpu.sync_copy(dst_hbm.at[wid], dst_v)
    pltpu.sync_copy(ones_hbm, ones_v)

    @pl.when(s == 0)
    def _():
        pltpu.sync_copy(zeros_hbm, acc_sh)

    plsc.subcore_barrier()

    def step(j, _):
        pltpu.sync_copy(ones_v, acc_sh.at[dst_v.at[j]], add=True)
        return ()

    lax.fori_loop(0, K, step, ())
    plsc.subcore_barrier()

    rows_per = NP // NS
    pltpu.sync_copy(acc_sh.at[pl.ds(s * rows_per, rows_per)],
                    out_hbm.at[c].at[pl.ds(s * rows_per, rows_per)])


@functools.partial(
    pl.kernel,
    out_type=jax.ShapeDtypeStruct((NC, NP, H), jnp.float32),
    mesh=_mesh,
    compiler_params=pltpu.CompilerParams(use_tc_tiling_on_sc=False),
    scratch_types=[
        pltpu.VMEM((K, CHUNK), jnp.int32),
        pltpu.VMEM((K, CHUNK), jnp.int32),
        pltpu.VMEM((NB, CHUNK, H), jnp.float32),
        pltpu.VMEM_SHARED((NP, H), jnp.float32),
        pltpu.SemaphoreType.DMA,
        pltpu.SemaphoreType.DMA,
        pltpu.SemaphoreType.DMA,
        pltpu.SemaphoreType.DMA,
    ],
)
def _sc_agg1(src_hbm, dst_hbm, p1_hbm, zeros_hbm, out_hbm,
             src_v, dst_v, rows_v, acc_sh, sem0, sem1, sem2, sem3):
    c = lax.axis_index("c")
    s = lax.axis_index("s")
    wid = c * NS + s
    sems = [sem0, sem1, sem2, sem3]
    pltpu.sync_copy(src_hbm.at[wid], src_v)
    pltpu.sync_copy(dst_hbm.at[wid], dst_v)

    @pl.when(s == 0)
    def _():
        pltpu.sync_copy(zeros_hbm, acc_sh)

    plsc.subcore_barrier()

    for b in range(NB):
        pltpu.async_copy(p1_hbm.at[src_v.at[b]], rows_v.at[b], sems[b])

    def outer(g, _):
        for b in range(NB):
            j = g * NB + b
            pltpu.make_async_copy(p1_hbm.at[src_v.at[j]],
                                  rows_v.at[b], sems[b]).wait()
            pltpu.sync_copy(rows_v.at[b], acc_sh.at[dst_v.at[j]], add=True)
            nj = j + NB

            @pl.when(nj < K)
            def _():
                pltpu.async_copy(p1_hbm.at[src_v.at[nj]], rows_v.at[b],
                                 sems[b])
        return ()

    lax.fori_loop(0, K // NB, outer, ())
    plsc.subcore_barrier()

    rows_per = NP // NS
    pltpu.sync_copy(acc_sh.at[pl.ds(s * rows_per, rows_per)],
                    out_hbm.at[c].at[pl.ds(s * rows_per, rows_per)])


@functools.partial(
    pl.kernel,
    out_type=jax.ShapeDtypeStruct((NC, NP), jnp.float32),
    mesh=_mesh,
    compiler_params=pltpu.CompilerParams(use_tc_tiling_on_sc=False),
    scratch_types=[
        pltpu.VMEM((K, CHUNK), jnp.int32),
        pltpu.VMEM((K, CHUNK), jnp.int32),
        pltpu.VMEM((NB, CHUNK), jnp.float32),
        pltpu.VMEM_SHARED((NP,), jnp.float32),
        pltpu.SemaphoreType.DMA,
        pltpu.SemaphoreType.DMA,
        pltpu.SemaphoreType.DMA,
        pltpu.SemaphoreType.DMA,
    ],
)
def _sc_agg2(src_hbm, dst_hbm, p2_hbm, zeros_hbm, out_hbm,
             src_v, dst_v, rows_v, acc_sh, sem0, sem1, sem2, sem3):
    c = lax.axis_index("c")
    s = lax.axis_index("s")
    wid = c * NS + s
    sems = [sem0, sem1, sem2, sem3]
    pltpu.sync_copy(src_hbm.at[wid], src_v)
    pltpu.sync_copy(dst_hbm.at[wid], dst_v)

    @pl.when(s == 0)
    def _():
        pltpu.sync_copy(zeros_hbm, acc_sh)

    plsc.subcore_barrier()

    for b in range(NB):
        pltpu.async_copy(p2_hbm.at[src_v.at[b]], rows_v.at[b], sems[b])

    def outer(g, _):
        for b in range(NB):
            j = g * NB + b
            pltpu.make_async_copy(p2_hbm.at[src_v.at[j]],
                                  rows_v.at[b], sems[b]).wait()
            pltpu.sync_copy(rows_v.at[b], acc_sh.at[dst_v.at[j]], add=True)
            nj = j + NB

            @pl.when(nj < K)
            def _():
                pltpu.async_copy(p2_hbm.at[src_v.at[nj]], rows_v.at[b],
                                 sems[b])
        return ()

    lax.fori_loop(0, K // NB, outer, ())
    plsc.subcore_barrier()

    rows_per = NP // NS
    pltpu.sync_copy(acc_sh.at[pl.ds(s * rows_per, rows_per)],
                    out_hbm.at[c].at[pl.ds(s * rows_per, rows_per)])


def _tc_a_body(deg_ref, x_ref, w1_ref, p1_ref, d_ref):
    deg = jnp.sum(deg_ref[...], axis=1, keepdims=True) + 1.0
    d = lax.rsqrt(deg)
    d_ref[...] = d
    p1_ref[...] = jnp.dot(x_ref[...], w1_ref[...],
                          preferred_element_type=jnp.float32) * d


def _tc_b_body(s1_ref, p1_ref, d_ref, b1_ref, w2_ref, p2_ref):
    s1 = s1_ref[0] + s1_ref[1] + p1_ref[...]
    h = jnp.maximum(s1 * d_ref[...] + b1_ref[...], 0.0)
    p2_ref[...] = jnp.dot(h, w2_ref[...],
                          preferred_element_type=jnp.float32) * d_ref[...]


def _tc_c_body(s2_ref, p2_ref, d_ref, b2_ref, out_ref):
    s2 = jnp.sum(s2_ref[...], axis=1, keepdims=True) + p2_ref[...]
    out_ref[...] = s2 * d_ref[...] + b2_ref[...]


def kernel(x, edge_index, W1, b1, W2, b2):
    src = edge_index[0]
    dst = edge_index[1]
    pad = EP - E
    padv = jnp.full((pad,), N, jnp.int32)
    src_p = jnp.concatenate([src, padv]).reshape(NW, K, CHUNK)
    dst_p = jnp.concatenate([dst, padv]).reshape(NW, K, CHUNK)
    x_p = jnp.pad(x, ((0, NP - N), (0, 0)))

    degT = _sc_deg(dst_p, jnp.ones((CHUNK,), jnp.float32),
                   jnp.zeros((NP,), jnp.float32))

    p1, d = pl.pallas_call(
        _tc_a_body,
        out_shape=(jax.ShapeDtypeStruct((NP, H), jnp.float32),
                   jax.ShapeDtypeStruct((NP, 1), jnp.float32)),
    )(degT.T, x_p, W1)

    s1p = _sc_agg1(src_p, dst_p, p1, jnp.zeros((NP, H), jnp.float32))

    p2col = pl.pallas_call(
        _tc_b_body,
        out_shape=jax.ShapeDtypeStruct((NP, 1), jnp.float32),
    )(s1p, p1, d, b1.reshape(1, H), W2)

    s2T = _sc_agg2(src_p, dst_p, p2col.reshape(NP),
                   jnp.zeros((NP,), jnp.float32))

    out = pl.pallas_call(
        _tc_c_body,
        out_shape=jax.ShapeDtypeStruct((NP, 1), jnp.float32),
    )(s2T.T, p2col, d, b2.reshape(1, 1))

    return out[:N, 0]


# speedup vs baseline: 60.8890x; 1.6431x over previous
import functools

import jax
import jax.numpy as jnp
from jax import lax
from jax.experimental import pallas as pl
from jax.experimental.pallas import tpu as pltpu
from jax.experimental.pallas import tpu_sc as plsc

NC = 2
NS = 16
NW = NC * NS

N = 10000
E = 320000
D = 128
H = 32

CHUNK = 128
K = 80
T = K * CHUNK
EP = NW * T
NP = 10240
NB = 4

_mesh = plsc.VectorSubcoreMesh(core_axis_name="c", subcore_axis_name="s")


@functools.partial(
    pl.kernel,
    out_type=jax.ShapeDtypeStruct((NC, NP), jnp.float32),
    mesh=_mesh,
    compiler_params=pltpu.CompilerParams(use_tc_tiling_on_sc=False),
    scratch_types=[
        pltpu.VMEM((K, CHUNK), jnp.int32),
        pltpu.VMEM((CHUNK,), jnp.float32),
        pltpu.VMEM_SHARED((NP,), jnp.float32),
    ],
)
def _sc_deg(dst_hbm, ones_hbm, zeros_hbm, out_hbm, dst_v, ones_v, acc_sh):
    c = lax.axis_index("c")
    s = lax.axis_index("s")
    wid = c * NS + s
    rp = NP // NS
    pltpu.sync_copy(dst_hbm.at[wid], dst_v)
    pltpu.sync_copy(ones_hbm, ones_v)
    pltpu.sync_copy(zeros_hbm.at[pl.ds(s * rp, rp)],
                    acc_sh.at[pl.ds(s * rp, rp)])
    plsc.subcore_barrier()

    def step(j, _):
        pltpu.sync_copy(ones_v, acc_sh.at[dst_v.at[j]], add=True)
        return ()

    lax.fori_loop(0, K, step, ())
    plsc.subcore_barrier()

    rows_per = NP // NS
    pltpu.sync_copy(acc_sh.at[pl.ds(s * rows_per, rows_per)],
                    out_hbm.at[c].at[pl.ds(s * rows_per, rows_per)])


@functools.partial(
    pl.kernel,
    out_type=jax.ShapeDtypeStruct((NC, NP, H), jnp.float32),
    mesh=_mesh,
    compiler_params=pltpu.CompilerParams(use_tc_tiling_on_sc=False),
    scratch_types=[
        pltpu.VMEM((K, CHUNK), jnp.int32),
        pltpu.VMEM((K, CHUNK), jnp.int32),
        pltpu.VMEM((NB, CHUNK, H), jnp.float32),
        pltpu.VMEM_SHARED((NP, H), jnp.float32),
        pltpu.VMEM_SHARED((NP, H), jnp.float32),
        pltpu.SemaphoreType.DMA,
        pltpu.SemaphoreType.DMA,
        pltpu.SemaphoreType.DMA,
        pltpu.SemaphoreType.DMA,
    ],
)
def _sc_agg1(src_hbm, dst_hbm, p1_hbm, zeros_hbm, out_hbm,
             src_v, dst_v, rows_v, p1_sh, acc_sh, sem0, sem1, sem2, sem3):
    c = lax.axis_index("c")
    s = lax.axis_index("s")
    wid = c * NS + s
    rp = NP // NS
    sems = [sem0, sem1, sem2, sem3]
    pltpu.sync_copy(src_hbm.at[wid], src_v)
    pltpu.sync_copy(dst_hbm.at[wid], dst_v)
    pltpu.sync_copy(p1_hbm.at[pl.ds(s * rp, rp)],
                    p1_sh.at[pl.ds(s * rp, rp)])
    pltpu.sync_copy(zeros_hbm.at[pl.ds(s * rp, rp)],
                    acc_sh.at[pl.ds(s * rp, rp)])
    plsc.subcore_barrier()

    for b in range(NB):
        pltpu.async_copy(p1_sh.at[src_v.at[b]], rows_v.at[b], sems[b])

    def outer(g, _):
        for b in range(NB):
            j = g * NB + b
            pltpu.make_async_copy(p1_sh.at[src_v.at[j]],
                                  rows_v.at[b], sems[b]).wait()
            pltpu.sync_copy(rows_v.at[b], acc_sh.at[dst_v.at[j]], add=True)
            nj = j + NB

            @pl.when(nj < K)
            def _():
                pltpu.async_copy(p1_sh.at[src_v.at[nj]], rows_v.at[b],
                                 sems[b])
        return ()

    lax.fori_loop(0, K // NB, outer, ())
    plsc.subcore_barrier()

    rows_per = NP // NS
    pltpu.sync_copy(acc_sh.at[pl.ds(s * rows_per, rows_per)],
                    out_hbm.at[c].at[pl.ds(s * rows_per, rows_per)])


@functools.partial(
    pl.kernel,
    out_type=jax.ShapeDtypeStruct((NC, NP), jnp.float32),
    mesh=_mesh,
    compiler_params=pltpu.CompilerParams(use_tc_tiling_on_sc=False),
    scratch_types=[
        pltpu.VMEM((K, CHUNK), jnp.int32),
        pltpu.VMEM((K, CHUNK), jnp.int32),
        pltpu.VMEM((NB, CHUNK), jnp.float32),
        pltpu.VMEM_SHARED((NP,), jnp.float32),
        pltpu.VMEM_SHARED((NP,), jnp.float32),
        pltpu.SemaphoreType.DMA,
        pltpu.SemaphoreType.DMA,
        pltpu.SemaphoreType.DMA,
        pltpu.SemaphoreType.DMA,
    ],
)
def _sc_agg2(src_hbm, dst_hbm, p2_hbm, zeros_hbm, out_hbm,
             src_v, dst_v, rows_v, p2_sh, acc_sh, sem0, sem1, sem2, sem3):
    c = lax.axis_index("c")
    s = lax.axis_index("s")
    wid = c * NS + s
    rp = NP // NS
    sems = [sem0, sem1, sem2, sem3]
    pltpu.sync_copy(src_hbm.at[wid], src_v)
    pltpu.sync_copy(dst_hbm.at[wid], dst_v)
    pltpu.sync_copy(p2_hbm.at[pl.ds(s * rp, rp)],
                    p2_sh.at[pl.ds(s * rp, rp)])
    pltpu.sync_copy(zeros_hbm.at[pl.ds(s * rp, rp)],
                    acc_sh.at[pl.ds(s * rp, rp)])
    plsc.subcore_barrier()

    for b in range(NB):
        pltpu.async_copy(p2_sh.at[src_v.at[b]], rows_v.at[b], sems[b])

    def outer(g, _):
        for b in range(NB):
            j = g * NB + b
            pltpu.make_async_copy(p2_sh.at[src_v.at[j]],
                                  rows_v.at[b], sems[b]).wait()
            pltpu.sync_copy(rows_v.at[b], acc_sh.at[dst_v.at[j]], add=True)
            nj = j + NB

            @pl.when(nj < K)
            def _():
                pltpu.async_copy(p2_sh.at[src_v.at[nj]], rows_v.at[b],
                                 sems[b])
        return ()

    lax.fori_loop(0, K // NB, outer, ())
    plsc.subcore_barrier()

    rows_per = NP // NS
    pltpu.sync_copy(acc_sh.at[pl.ds(s * rows_per, rows_per)],
                    out_hbm.at[c].at[pl.ds(s * rows_per, rows_per)])


def _tc_a_body(deg_ref, x_ref, w1_ref, p1_ref, d_ref):
    deg = jnp.sum(deg_ref[...], axis=1, keepdims=True) + 1.0
    d = lax.rsqrt(deg)
    d_ref[...] = d
    p1_ref[...] = jnp.dot(x_ref[...], w1_ref[...],
                          preferred_element_type=jnp.float32) * d


def _tc_b_body(s1_ref, p1_ref, d_ref, b1_ref, w2_ref, p2_ref):
    s1 = s1_ref[0] + s1_ref[1] + p1_ref[...]
    h = jnp.maximum(s1 * d_ref[...] + b1_ref[...], 0.0)
    p2_ref[...] = jnp.dot(h, w2_ref[...],
                          preferred_element_type=jnp.float32) * d_ref[...]


def _tc_c_body(s2_ref, p2_ref, d_ref, b2_ref, out_ref):
    s2 = jnp.sum(s2_ref[...], axis=1, keepdims=True) + p2_ref[...]
    out_ref[...] = s2 * d_ref[...] + b2_ref[...]


def kernel(x, edge_index, W1, b1, W2, b2):
    src = edge_index[0]
    dst = edge_index[1]
    pad = EP - E
    padv = jnp.full((pad,), N, jnp.int32)
    src_p = jnp.concatenate([src, padv]).reshape(NW, K, CHUNK)
    dst_p = jnp.concatenate([dst, padv]).reshape(NW, K, CHUNK)
    x_p = jnp.pad(x, ((0, NP - N), (0, 0)))

    degT = _sc_deg(dst_p, jnp.ones((CHUNK,), jnp.float32),
                   jnp.zeros((NP,), jnp.float32))

    p1, d = pl.pallas_call(
        _tc_a_body,
        out_shape=(jax.ShapeDtypeStruct((NP, H), jnp.float32),
                   jax.ShapeDtypeStruct((NP, 1), jnp.float32)),
    )(degT.T, x_p, W1)

    s1p = _sc_agg1(src_p, dst_p, p1, jnp.zeros((NP, H), jnp.float32))

    p2col = pl.pallas_call(
        _tc_b_body,
        out_shape=jax.ShapeDtypeStruct((NP, 1), jnp.float32),
    )(s1p, p1, d, b1.reshape(1, H), W2)

    s2T = _sc_agg2(src_p, dst_p, p2col.reshape(NP),
                   jnp.zeros((NP,), jnp.float32))

    out = pl.pallas_call(
        _tc_c_body,
        out_shape=jax.ShapeDtypeStruct((NP, 1), jnp.float32),
    )(s2T.T, p2col, d, b2.reshape(1, 1))

    return out[:N, 0]


# speedup vs baseline: 61.1730x; 1.0047x over previous
import functools

import jax
import jax.numpy as jnp
from jax import lax
from jax.experimental import pallas as pl
from jax.experimental.pallas import tpu as pltpu
from jax.experimental.pallas import tpu_sc as plsc

NC = 2
NS = 16
NW = NC * NS

N = 10000
E = 320000
D = 128
H = 32

CHUNK = 128
K = 80
T = K * CHUNK
EP = NW * T
NP = 10240
NB = 4

_mesh = plsc.VectorSubcoreMesh(core_axis_name="c", subcore_axis_name="s")


@functools.partial(
    pl.kernel,
    out_type=jax.ShapeDtypeStruct((NC, NP), jnp.float32),
    mesh=_mesh,
    compiler_params=pltpu.CompilerParams(use_tc_tiling_on_sc=False),
    scratch_types=[
        pltpu.VMEM((K, CHUNK), jnp.int32),
        pltpu.VMEM((CHUNK,), jnp.float32),
        pltpu.VMEM_SHARED((NP,), jnp.float32),
    ],
)
def _sc_deg(dst_hbm, ones_hbm, zeros_hbm, out_hbm, dst_v, ones_v, acc_sh):
    c = lax.axis_index("c")
    s = lax.axis_index("s")
    wid = c * NS + s
    rp = NP // NS
    pltpu.sync_copy(dst_hbm.at[wid], dst_v)
    pltpu.sync_copy(ones_hbm, ones_v)
    pltpu.sync_copy(zeros_hbm.at[pl.ds(s * rp, rp)],
                    acc_sh.at[pl.ds(s * rp, rp)])
    plsc.subcore_barrier()

    def step(j, _):
        pltpu.sync_copy(ones_v, acc_sh.at[dst_v.at[j]], add=True)
        return ()

    lax.fori_loop(0, K, step, ())
    plsc.subcore_barrier()

    rows_per = NP // NS
    pltpu.sync_copy(acc_sh.at[pl.ds(s * rows_per, rows_per)],
                    out_hbm.at[c].at[pl.ds(s * rows_per, rows_per)])


@functools.partial(
    pl.kernel,
    out_type=jax.ShapeDtypeStruct((NC, NP, H), jnp.float32),
    mesh=_mesh,
    compiler_params=pltpu.CompilerParams(use_tc_tiling_on_sc=False),
    scratch_types=[
        pltpu.VMEM((K, CHUNK), jnp.int32),
        pltpu.VMEM((K, CHUNK), jnp.int32),
        pltpu.VMEM((NB, CHUNK, H), jnp.float32),
        pltpu.VMEM_SHARED((NP, H), jnp.float32),
        pltpu.VMEM_SHARED((NP, H), jnp.float32),
        pltpu.SemaphoreType.DMA,
        pltpu.SemaphoreType.DMA,
        pltpu.SemaphoreType.DMA,
        pltpu.SemaphoreType.DMA,
    ],
)
def _sc_agg1(src_hbm, dst_hbm, p1_hbm, zeros_hbm, out_hbm,
             src_v, dst_v, rows_v, p1_sh, acc_sh, sem0, sem1, sem2, sem3):
    c = lax.axis_index("c")
    s = lax.axis_index("s")
    wid = c * NS + s
    rp = NP // NS
    sems = [sem0, sem1, sem2, sem3]
    pltpu.sync_copy(src_hbm.at[wid], src_v)
    pltpu.sync_copy(dst_hbm.at[wid], dst_v)
    pltpu.sync_copy(p1_hbm.at[pl.ds(s * rp, rp)],
                    p1_sh.at[pl.ds(s * rp, rp)])
    pltpu.sync_copy(zeros_hbm.at[pl.ds(s * rp, rp)],
                    acc_sh.at[pl.ds(s * rp, rp)])
    plsc.subcore_barrier()

    for b in range(NB):
        pltpu.async_copy(p1_sh.at[src_v.at[b]], rows_v.at[b], sems[b])

    def outer(g, _):
        for b in range(NB):
            j = g * NB + b
            pltpu.make_async_copy(p1_sh.at[src_v.at[j]],
                                  rows_v.at[b], sems[b]).wait()
            pltpu.sync_copy(rows_v.at[b], acc_sh.at[dst_v.at[j]], add=True)
            nj = j + NB

            @pl.when(nj < K)
            def _():
                pltpu.async_copy(p1_sh.at[src_v.at[nj]], rows_v.at[b],
                                 sems[b])
        return ()

    lax.fori_loop(0, K // NB, outer, ())
    plsc.subcore_barrier()

    rows_per = NP // NS
    pltpu.sync_copy(acc_sh.at[pl.ds(s * rows_per, rows_per)],
                    out_hbm.at[c].at[pl.ds(s * rows_per, rows_per)])


@functools.partial(
    pl.kernel,
    out_type=jax.ShapeDtypeStruct((NC, NP), jnp.float32),
    mesh=_mesh,
    compiler_params=pltpu.CompilerParams(use_tc_tiling_on_sc=False),
    scratch_types=[
        pltpu.VMEM((K, CHUNK), jnp.int32),
        pltpu.VMEM((K, CHUNK), jnp.int32),
        pltpu.VMEM((NB, CHUNK), jnp.float32),
        pltpu.VMEM_SHARED((NP,), jnp.float32),
        pltpu.VMEM_SHARED((NP,), jnp.float32),
        pltpu.SemaphoreType.DMA,
        pltpu.SemaphoreType.DMA,
        pltpu.SemaphoreType.DMA,
        pltpu.SemaphoreType.DMA,
    ],
)
def _sc_agg2(src_hbm, dst_hbm, p2_hbm, zeros_hbm, out_hbm,
             src_v, dst_v, rows_v, p2_sh, acc_sh, sem0, sem1, sem2, sem3):
    c = lax.axis_index("c")
    s = lax.axis_index("s")
    wid = c * NS + s
    rp = NP // NS
    sems = [sem0, sem1, sem2, sem3]
    pltpu.sync_copy(src_hbm.at[wid], src_v)
    pltpu.sync_copy(dst_hbm.at[wid], dst_v)
    pltpu.sync_copy(p2_hbm.at[pl.ds(s * rp, rp)],
                    p2_sh.at[pl.ds(s * rp, rp)])
    pltpu.sync_copy(zeros_hbm.at[pl.ds(s * rp, rp)],
                    acc_sh.at[pl.ds(s * rp, rp)])
    plsc.subcore_barrier()

    for b in range(NB):
        pltpu.async_copy(p2_sh.at[src_v.at[b]], rows_v.at[b], sems[b])

    def outer(g, _):
        for b in range(NB):
            j = g * NB + b
            pltpu.make_async_copy(p2_sh.at[src_v.at[j]],
                                  rows_v.at[b], sems[b]).wait()
            pltpu.sync_copy(rows_v.at[b], acc_sh.at[dst_v.at[j]], add=True)
            nj = j + NB

            @pl.when(nj < K)
            def _():
                pltpu.async_copy(p2_sh.at[src_v.at[nj]], rows_v.at[b],
                                 sems[b])
        return ()

    lax.fori_loop(0, K // NB, outer, ())
    plsc.subcore_barrier()

    rows_per = NP // NS
    pltpu.sync_copy(acc_sh.at[pl.ds(s * rows_per, rows_per)],
                    out_hbm.at[c].at[pl.ds(s * rows_per, rows_per)])


def _tc_q_body(x_ref, w1_ref, q1_ref):
    q1_ref[...] = jnp.dot(x_ref[...], w1_ref[...],
                          preferred_element_type=jnp.float32)


def _tc_a_body(deg_ref, q1_ref, p1_ref, d_ref):
    deg = jnp.sum(deg_ref[...], axis=1, keepdims=True) + 1.0
    d = lax.rsqrt(deg)
    d_ref[...] = d
    p1_ref[...] = q1_ref[...] * d


def _tc_b_body(s1_ref, p1_ref, d_ref, b1_ref, w2_ref, p2_ref):
    s1 = s1_ref[0] + s1_ref[1] + p1_ref[...]
    h = jnp.maximum(s1 * d_ref[...] + b1_ref[...], 0.0)
    p2_ref[...] = jnp.dot(h, w2_ref[...],
                          preferred_element_type=jnp.float32) * d_ref[...]


def _tc_c_body(s2_ref, p2_ref, d_ref, b2_ref, out_ref):
    s2 = jnp.sum(s2_ref[...], axis=1, keepdims=True) + p2_ref[...]
    out_ref[...] = s2 * d_ref[...] + b2_ref[...]


def kernel(x, edge_index, W1, b1, W2, b2):
    src = edge_index[0]
    dst = edge_index[1]
    pad = EP - E
    padv = jnp.full((pad,), N, jnp.int32)
    src_p = jnp.concatenate([src, padv]).reshape(NW, K, CHUNK)
    dst_p = jnp.concatenate([dst, padv]).reshape(NW, K, CHUNK)
    x_p = jnp.pad(x, ((0, NP - N), (0, 0)))

    degT = _sc_deg(dst_p, jnp.ones((CHUNK,), jnp.float32),
                   jnp.zeros((NP,), jnp.float32))

    q1 = pl.pallas_call(
        _tc_q_body,
        out_shape=jax.ShapeDtypeStruct((NP, H), jnp.float32),
    )(x_p, W1)

    p1, d = pl.pallas_call(
        _tc_a_body,
        out_shape=(jax.ShapeDtypeStruct((NP, H), jnp.float32),
                   jax.ShapeDtypeStruct((NP, 1), jnp.float32)),
    )(degT.T, q1)

    s1p = _sc_agg1(src_p, dst_p, p1, jnp.zeros((NP, H), jnp.float32))

    p2col = pl.pallas_call(
        _tc_b_body,
        out_shape=jax.ShapeDtypeStruct((NP, 1), jnp.float32),
    )(s1p, p1, d, b1.reshape(1, H), W2)

    s2T = _sc_agg2(src_p, dst_p, p2col.reshape(NP),
                   jnp.zeros((NP,), jnp.float32))

    out = pl.pallas_call(
        _tc_c_body,
        out_shape=jax.ShapeDtypeStruct((NP, 1), jnp.float32),
    )(s2T.T, p2col, d, b2.reshape(1, 1))

    return out[:N, 0]


# speedup vs baseline: 67.0449x; 1.0960x over previous
import functools

import jax
import jax.numpy as jnp
from jax import lax
from jax.experimental import pallas as pl
from jax.experimental.pallas import tpu as pltpu
from jax.experimental.pallas import tpu_sc as plsc

NC = 2
NS = 16
NW = NC * NS

N = 10000
E = 320000
D = 128
H = 32

CHUNK = 128
K = 84
T = K * CHUNK
EP = NW * T
NP = 10240
NPK = NP // 128
NPH = NP * H // 128
NB = 4

_mesh = plsc.VectorSubcoreMesh(core_axis_name="c", subcore_axis_name="s")


@functools.partial(
    pl.kernel,
    out_type=jax.ShapeDtypeStruct((NC, NP), jnp.float32),
    mesh=_mesh,
    compiler_params=pltpu.CompilerParams(use_tc_tiling_on_sc=False),
    scratch_types=[
        pltpu.VMEM((K, CHUNK), jnp.int32),
        pltpu.VMEM((CHUNK,), jnp.float32),
        pltpu.VMEM_SHARED((NP,), jnp.float32),
    ],
)
def _sc_deg(dst_hbm, ones_hbm, zeros_hbm, out_hbm, dst_v, ones_v, acc_sh):
    c = lax.axis_index("c")
    s = lax.axis_index("s")
    wid = c * NS + s
    rp = NP // NS
    pltpu.sync_copy(dst_hbm.at[wid], dst_v)
    pltpu.sync_copy(ones_hbm, ones_v)
    pltpu.sync_copy(zeros_hbm.at[pl.ds(s * rp, rp)],
                    acc_sh.at[pl.ds(s * rp, rp)])
    plsc.subcore_barrier()

    def step(j, _):
        pltpu.sync_copy(ones_v, acc_sh.at[dst_v.at[j]], add=True)
        return ()

    lax.fori_loop(0, K, step, ())
    plsc.subcore_barrier()

    pltpu.sync_copy(acc_sh.at[pl.ds(s * rp, rp)],
                    out_hbm.at[c].at[pl.ds(s * rp, rp)])


@functools.partial(
    pl.kernel,
    out_type=jax.ShapeDtypeStruct((NC, NP, H), jnp.float32),
    mesh=_mesh,
    compiler_params=pltpu.CompilerParams(use_tc_tiling_on_sc=False),
    scratch_types=[
        pltpu.VMEM((K, CHUNK), jnp.int32),
        pltpu.VMEM((K, CHUNK), jnp.int32),
        pltpu.VMEM((NB, CHUNK, H), jnp.float32),
        pltpu.VMEM_SHARED((NP, H), jnp.float32),
        pltpu.VMEM_SHARED((NP, H), jnp.float32),
        pltpu.SemaphoreType.DMA,
        pltpu.SemaphoreType.DMA,
        pltpu.SemaphoreType.DMA,
        pltpu.SemaphoreType.DMA,
    ],
)
def _sc_agg1(src_hbm, dst_hbm, p1_hbm, zeros_hbm, out_hbm,
             src_v, dst_v, rows_v, p1_sh, acc_sh, sem0, sem1, sem2, sem3):
    c = lax.axis_index("c")
    s = lax.axis_index("s")
    wid = c * NS + s
    rp = NP // NS
    sems = [sem0, sem1, sem2, sem3]
    pltpu.sync_copy(src_hbm.at[wid], src_v)
    pltpu.sync_copy(dst_hbm.at[wid], dst_v)
    pltpu.sync_copy(p1_hbm.at[pl.ds(s * rp, rp)],
                    p1_sh.at[pl.ds(s * rp, rp)])
    pltpu.sync_copy(zeros_hbm.at[pl.ds(s * rp, rp)],
                    acc_sh.at[pl.ds(s * rp, rp)])
    plsc.subcore_barrier()

    for b in range(NB):
        pltpu.async_copy(p1_sh.at[src_v.at[b]], rows_v.at[b], sems[b])

    def outer(g, _):
        for b in range(NB):
            j = g * NB + b
            pltpu.make_async_copy(p1_sh.at[src_v.at[j]],
                                  rows_v.at[b], sems[b]).wait()
            pltpu.sync_copy(rows_v.at[b], acc_sh.at[dst_v.at[j]], add=True)
            nj = j + NB

            @pl.when(nj < K)
            def _():
                pltpu.async_copy(p1_sh.at[src_v.at[nj]], rows_v.at[b],
                                 sems[b])
        return ()

    lax.fori_loop(0, K // NB, outer, ())
    plsc.subcore_barrier()

    pltpu.sync_copy(acc_sh.at[pl.ds(s * rp, rp)],
                    out_hbm.at[c].at[pl.ds(s * rp, rp)])


@functools.partial(
    pl.kernel,
    out_type=jax.ShapeDtypeStruct((NC, NP), jnp.float32),
    mesh=_mesh,
    compiler_params=pltpu.CompilerParams(use_tc_tiling_on_sc=False),
    scratch_types=[
        pltpu.VMEM((K, CHUNK), jnp.int32),
        pltpu.VMEM((K, CHUNK), jnp.int32),
        pltpu.VMEM((NB, CHUNK), jnp.float32),
        pltpu.VMEM_SHARED((NP,), jnp.float32),
        pltpu.VMEM_SHARED((NP,), jnp.float32),
        pltpu.SemaphoreType.DMA,
        pltpu.SemaphoreType.DMA,
        pltpu.SemaphoreType.DMA,
        pltpu.SemaphoreType.DMA,
    ],
)
def _sc_agg2(src_hbm, dst_hbm, p2_hbm, zeros_hbm, out_hbm,
             src_v, dst_v, rows_v, p2_sh, acc_sh, sem0, sem1, sem2, sem3):
    c = lax.axis_index("c")
    s = lax.axis_index("s")
    wid = c * NS + s
    rp = NP // NS
    sems = [sem0, sem1, sem2, sem3]
    pltpu.sync_copy(src_hbm.at[wid], src_v)
    pltpu.sync_copy(dst_hbm.at[wid], dst_v)
    pltpu.sync_copy(p2_hbm.at[pl.ds(s * rp, rp)],
                    p2_sh.at[pl.ds(s * rp, rp)])
    pltpu.sync_copy(zeros_hbm.at[pl.ds(s * rp, rp)],
                    acc_sh.at[pl.ds(s * rp, rp)])
    plsc.subcore_barrier()

    for b in range(NB):
        pltpu.async_copy(p2_sh.at[src_v.at[b]], rows_v.at[b], sems[b])

    def outer(g, _):
        for b in range(NB):
            j = g * NB + b
            pltpu.make_async_copy(p2_sh.at[src_v.at[j]],
                                  rows_v.at[b], sems[b]).wait()
            pltpu.sync_copy(rows_v.at[b], acc_sh.at[dst_v.at[j]], add=True)
            nj = j + NB

            @pl.when(nj < K)
            def _():
                pltpu.async_copy(p2_sh.at[src_v.at[nj]], rows_v.at[b],
                                 sems[b])
        return ()

    lax.fori_loop(0, K // NB, outer, ())
    plsc.subcore_barrier()

    pltpu.sync_copy(acc_sh.at[pl.ds(s * rp, rp)],
                    out_hbm.at[c].at[pl.ds(s * rp, rp)])


def _tc_a_body(degexp_ref, degpk_ref, x_ref, w1blk_ref,
               p1pk_ref, dexp_ref, dpk_ref):
    dexp = lax.rsqrt(jnp.maximum(degexp_ref[...], 1.0))
    dexp_ref[...] = dexp
    dpk_ref[...] = lax.rsqrt(
        jnp.maximum(degpk_ref[0] + degpk_ref[1], 1.0))
    u = jnp.dot(x_ref[pl.ds(0, NPH), :], w1blk_ref[pl.ds(0, D), :],
                preferred_element_type=jnp.float32)
    for a in range(1, 4):
        u += jnp.dot(x_ref[pl.ds(a * NPH, NPH), :],
                     w1blk_ref[pl.ds(a * D, D), :],
                     preferred_element_type=jnp.float32)
    p1pk_ref[...] = u * dexp


def _tc_b_body(s1pk_ref, dexp_ref, b1t_ref, m_ref, p2pk_ref):
    s1 = s1pk_ref[0] + s1pk_ref[1]
    h = jnp.maximum(s1 * dexp_ref[...] + b1t_ref[...], 0.0)
    p2pk_ref[...] = jnp.dot(h, m_ref[...],
                            preferred_element_type=jnp.float32) * dexp_ref[...]


def _tc_c_body(s2pk_ref, dpk_ref, b2_ref, out_ref):
    s2 = s2pk_ref[0] + s2pk_ref[1]
    out_ref[...] = s2 * dpk_ref[...] + b2_ref[...]


def kernel(x, edge_index, W1, b1, W2, b2):
    src = edge_index[0]
    dst = edge_index[1]
    pad = EP - E - NP
    loop = jnp.arange(NP, dtype=jnp.int32)
    padv = jnp.full((pad,), N, jnp.int32)
    src_a = jnp.concatenate([src, loop, padv])
    dst_a = jnp.concatenate([dst, loop, padv])
    src_p = ((src_a % NPH) * 4 + src_a // NPH).reshape(NW, K, CHUNK)
    dst_p = ((dst_a % NPH) * 4 + dst_a // NPH).reshape(NW, K, CHUNK)
    x_p = jnp.pad(x, ((0, NP - N), (0, 0)))

    ones_c = jnp.ones((CHUNK,), jnp.float32)
    zeros_n = jnp.zeros((NP,), jnp.float32)
    zeros_nh = jnp.zeros((NP, H), jnp.float32)

    degT = _sc_deg(dst_p, ones_c, zeros_n)
    degpk = degT.reshape(NC, NPK, CHUNK)
    deg_pi = degT[0] + degT[1]
    degexp = jnp.repeat(deg_pi.reshape(NPH, 4), H, axis=1)

    w1blk = jnp.kron(jnp.eye(4, dtype=jnp.float32), W1)

    p1pk, dexp, d_pk = pl.pallas_call(
        _tc_a_body,
        out_shape=(jax.ShapeDtypeStruct((NPH, CHUNK), jnp.float32),
                   jax.ShapeDtypeStruct((NPH, CHUNK), jnp.float32),
                   jax.ShapeDtypeStruct((NPK, CHUNK), jnp.float32)),
    )(degexp, degpk, x_p, w1blk)

    s1p = _sc_agg1(src_p, dst_p, p1pk.reshape(NP, H), zeros_nh)
    s1pk = s1p.reshape(NC, NPH, CHUNK)

    b1t = jnp.tile(b1, 4).reshape(1, CHUNK)
    m_blk = jnp.kron(jnp.eye(4, dtype=jnp.float32),
                     jnp.tile(W2, (1, H)))

    p2pk = pl.pallas_call(
        _tc_b_body,
        out_shape=jax.ShapeDtypeStruct((NPH, CHUNK), jnp.float32),
    )(s1pk, dexp, b1t, m_blk)

    p2_flat = p2pk.reshape(NP, H)[:, 0]

    s2T = _sc_agg2(src_p, dst_p, p2_flat, zeros_n)
    s2pk = s2T.reshape(NC, NPK, CHUNK)

    outpk = pl.pallas_call(
        _tc_c_body,
        out_shape=jax.ShapeDtypeStruct((NPK, CHUNK), jnp.float32),
    )(s2pk, d_pk, b2.reshape(1, 1))

    out_n = outpk.reshape(NPH, 4).T.reshape(NP)
    return out_n[:N]


# speedup vs baseline: 76.1044x; 1.1351x over previous
import functools

import jax
import jax.numpy as jnp
from jax import lax
from jax.experimental import pallas as pl
from jax.experimental.pallas import tpu as pltpu
from jax.experimental.pallas import tpu_sc as plsc

NC = 2
NS = 16
NW = NC * NS

N = 10000
E = 320000
D = 128
H = 32

CHUNK = 128
K = 81
T = K * CHUNK
EP = NW * T
NP = 10240
NPK = NP // 128
NPH = NP * H // 128
NB = 3

_mesh = plsc.VectorSubcoreMesh(core_axis_name="c", subcore_axis_name="s")


@functools.partial(
    pl.kernel,
    out_type=jax.ShapeDtypeStruct((NC, NP), jnp.float32),
    mesh=_mesh,
    compiler_params=pltpu.CompilerParams(use_tc_tiling_on_sc=False),
    scratch_types=[
        pltpu.VMEM((K, CHUNK), jnp.int32),
        pltpu.VMEM((CHUNK,), jnp.float32),
        pltpu.VMEM_SHARED((NP,), jnp.float32),
    ],
)
def _sc_deg(dst_hbm, ones_hbm, zeros_hbm, out_hbm, dst_v, ones_v, acc_sh):
    c = lax.axis_index("c")
    s = lax.axis_index("s")
    wid = c * NS + s
    rp = NP // NS
    pltpu.sync_copy(dst_hbm.at[wid], dst_v)
    pltpu.sync_copy(ones_hbm, ones_v)
    pltpu.sync_copy(zeros_hbm.at[pl.ds(s * rp, rp)],
                    acc_sh.at[pl.ds(s * rp, rp)])
    plsc.subcore_barrier()

    def step(j, _):
        pltpu.sync_copy(ones_v, acc_sh.at[dst_v.at[j]], add=True)
        return ()

    lax.fori_loop(0, K, step, ())
    plsc.subcore_barrier()

    pltpu.sync_copy(acc_sh.at[pl.ds(s * rp, rp)],
                    out_hbm.at[c].at[pl.ds(s * rp, rp)])


@functools.partial(
    pl.kernel,
    out_type=jax.ShapeDtypeStruct((NC, NP, H), jnp.float32),
    mesh=_mesh,
    compiler_params=pltpu.CompilerParams(use_tc_tiling_on_sc=False),
    scratch_types=[
        pltpu.VMEM((K, CHUNK), jnp.int32),
        pltpu.VMEM((K, CHUNK), jnp.int32),
        pltpu.VMEM((NB, CHUNK, H), jnp.float32),
        pltpu.VMEM_SHARED((NP, H), jnp.float32),
        pltpu.VMEM_SHARED((NP, H), jnp.float32),
        pltpu.SemaphoreType.DMA,
        pltpu.SemaphoreType.DMA,
        pltpu.SemaphoreType.DMA,
    ],
)
def _sc_agg1(src_hbm, dst_hbm, p1_hbm, zeros_hbm, out_hbm,
             src_v, dst_v, rows_v, p1_sh, acc_sh, sem0, sem1, sem2):
    c = lax.axis_index("c")
    s = lax.axis_index("s")
    wid = c * NS + s
    rp = NP // NS
    sems = [sem0, sem1, sem2]
    pltpu.sync_copy(src_hbm.at[wid], src_v)
    pltpu.sync_copy(dst_hbm.at[wid], dst_v)
    pltpu.sync_copy(p1_hbm.at[pl.ds(s * rp, rp)],
                    p1_sh.at[pl.ds(s * rp, rp)])
    pltpu.sync_copy(zeros_hbm.at[pl.ds(s * rp, rp)],
                    acc_sh.at[pl.ds(s * rp, rp)])
    plsc.subcore_barrier()

    for b in range(NB):
        pltpu.async_copy(p1_sh.at[src_v.at[b]], rows_v.at[b], sems[b])

    def outer(g, _):
        for b in range(NB):
            j = g * NB + b
            pltpu.make_async_copy(p1_sh.at[src_v.at[j]],
                                  rows_v.at[b], sems[b]).wait()
            pltpu.sync_copy(rows_v.at[b], acc_sh.at[dst_v.at[j]], add=True)
            nj = j + NB

            @pl.when(nj < K)
            def _():
                pltpu.async_copy(p1_sh.at[src_v.at[nj]], rows_v.at[b],
                                 sems[b])
        return ()

    lax.fori_loop(0, K // NB, outer, ())
    plsc.subcore_barrier()

    pltpu.sync_copy(acc_sh.at[pl.ds(s * rp, rp)],
                    out_hbm.at[c].at[pl.ds(s * rp, rp)])


@functools.partial(
    pl.kernel,
    out_type=jax.ShapeDtypeStruct((NC, NP), jnp.float32),
    mesh=_mesh,
    compiler_params=pltpu.CompilerParams(use_tc_tiling_on_sc=False),
    scratch_types=[
        pltpu.VMEM((K, CHUNK), jnp.int32),
        pltpu.VMEM((K, CHUNK), jnp.int32),
        pltpu.VMEM((NB, CHUNK), jnp.float32),
        pltpu.VMEM_SHARED((NP,), jnp.float32),
        pltpu.VMEM_SHARED((NP,), jnp.float32),
        pltpu.SemaphoreType.DMA,
        pltpu.SemaphoreType.DMA,
        pltpu.SemaphoreType.DMA,
    ],
)
def _sc_agg2(src_hbm, dst_hbm, p2_hbm, zeros_hbm, out_hbm,
             src_v, dst_v, rows_v, p2_sh, acc_sh, sem0, sem1, sem2):
    c = lax.axis_index("c")
    s = lax.axis_index("s")
    wid = c * NS + s
    rp = NP // NS
    sems = [sem0, sem1, sem2]
    pltpu.sync_copy(src_hbm.at[wid], src_v)
    pltpu.sync_copy(dst_hbm.at[wid], dst_v)
    pltpu.sync_copy(p2_hbm.at[pl.ds(s * rp, rp)],
                    p2_sh.at[pl.ds(s * rp, rp)])
    pltpu.sync_copy(zeros_hbm.at[pl.ds(s * rp, rp)],
                    acc_sh.at[pl.ds(s * rp, rp)])
    plsc.subcore_barrier()

    for b in range(NB):
        pltpu.async_copy(p2_sh.at[src_v.at[b]], rows_v.at[b], sems[b])

    def outer(g, _):
        for b in range(NB):
            j = g * NB + b
            pltpu.make_async_copy(p2_sh.at[src_v.at[j]],
                                  rows_v.at[b], sems[b]).wait()
            pltpu.sync_copy(rows_v.at[b], acc_sh.at[dst_v.at[j]], add=True)
            nj = j + NB

            @pl.when(nj < K)
            def _():
                pltpu.async_copy(p2_sh.at[src_v.at[nj]], rows_v.at[b],
                                 sems[b])
        return ()

    lax.fori_loop(0, K // NB, outer, ())
    plsc.subcore_barrier()

    pltpu.sync_copy(acc_sh.at[pl.ds(s * rp, rp)],
                    out_hbm.at[c].at[pl.ds(s * rp, rp)])


def _tc_a_body(degexp_ref, degpk_ref, x_ref, w1blk_ref,
               p1pk_ref, dexp_ref, dpk_ref):
    dexp = lax.rsqrt(jnp.maximum(degexp_ref[...], 1.0))
    dexp_ref[...] = dexp
    dpk_ref[...] = lax.rsqrt(
        jnp.maximum(degpk_ref[0] + degpk_ref[1], 1.0))
    u = jnp.dot(x_ref[pl.ds(0, NPH), :], w1blk_ref[pl.ds(0, D), :],
                preferred_element_type=jnp.float32)
    for a in range(1, 4):
        u += jnp.dot(x_ref[pl.ds(a * NPH, NPH), :],
                     w1blk_ref[pl.ds(a * D, D), :],
                     preferred_element_type=jnp.float32)
    p1pk_ref[...] = u * dexp


def _tc_b_body(s1pk_ref, dexp_ref, b1t_ref, m_ref, p2pk_ref):
    s1 = s1pk_ref[0] + s1pk_ref[1]
    h = jnp.maximum(s1 * dexp_ref[...] + b1t_ref[...], 0.0)
    p2pk_ref[...] = jnp.dot(h, m_ref[...],
                            preferred_element_type=jnp.float32) * dexp_ref[...]


def _tc_c_body(s2pk_ref, dpk_ref, b2_ref, out_ref):
    s2 = s2pk_ref[0] + s2pk_ref[1]
    out_ref[...] = s2 * dpk_ref[...] + b2_ref[...]


def kernel(x, edge_index, W1, b1, W2, b2):
    src = edge_index[0]
    dst = edge_index[1]
    pad = EP - E - NP
    loop = jnp.arange(NP, dtype=jnp.int32)
    padv = N + (jnp.arange(pad, dtype=jnp.int32) % (NP - N))
    src_a = jnp.concatenate([src, loop, padv])
    dst_a = jnp.concatenate([dst, loop, padv])
    src_p = ((src_a % NPH) * 4 + src_a // NPH).reshape(NW, K, CHUNK)
    dst_p = ((dst_a % NPH) * 4 + dst_a // NPH).reshape(NW, K, CHUNK)
    x_p = jnp.pad(x, ((0, NP - N), (0, 0)))

    ones_c = jnp.ones((CHUNK,), jnp.float32)
    zeros_n = jnp.zeros((NP,), jnp.float32)
    zeros_nh = jnp.zeros((NP, H), jnp.float32)

    degT = _sc_deg(dst_p, ones_c, zeros_n)
    degpk = degT.reshape(NC, NPK, CHUNK)
    deg_pi = degT[0] + degT[1]
    degexp = jnp.repeat(deg_pi.reshape(NPH, 4), H, axis=1)

    w1blk = jnp.kron(jnp.eye(4, dtype=jnp.float32), W1)

    p1pk, dexp, d_pk = pl.pallas_call(
        _tc_a_body,
        out_shape=(jax.ShapeDtypeStruct((NPH, CHUNK), jnp.float32),
                   jax.ShapeDtypeStruct((NPH, CHUNK), jnp.float32),
                   jax.ShapeDtypeStruct((NPK, CHUNK), jnp.float32)),
    )(degexp, degpk, x_p, w1blk)

    s1p = _sc_agg1(src_p, dst_p, p1pk.reshape(NP, H), zeros_nh)
    s1pk = s1p.reshape(NC, NPH, CHUNK)

    b1t = jnp.tile(b1, 4).reshape(1, CHUNK)
    m_blk = jnp.kron(jnp.eye(4, dtype=jnp.float32),
                     jnp.tile(W2, (1, H)))

    p2pk = pl.pallas_call(
        _tc_b_body,
        out_shape=jax.ShapeDtypeStruct((NPH, CHUNK), jnp.float32),
    )(s1pk, dexp, b1t, m_blk)

    p2_flat = p2pk.reshape(NP, H)[:, 0]

    s2T = _sc_agg2(src_p, dst_p, p2_flat, zeros_n)
    s2pk = s2T.reshape(NC, NPK, CHUNK)

    outpk = pl.pallas_call(
        _tc_c_body,
        out_shape=jax.ShapeDtypeStruct((NPK, CHUNK), jnp.float32),
    )(s2pk, d_pk, b2.reshape(1, 1))

    out_n = outpk.reshape(NPH, 4).T.reshape(NP)
    return out_n[:N]


# speedup vs baseline: 76.6657x; 1.0074x over previous
import functools

import jax
import jax.numpy as jnp
from jax import lax
from jax.experimental import pallas as pl
from jax.experimental.pallas import tpu as pltpu
from jax.experimental.pallas import tpu_sc as plsc

NC = 2
NS = 16
NW = NC * NS

N = 10000
E = 320000
D = 128
H = 32

CHUNK = 128
K = 81
T = K * CHUNK
EP = NW * T
NP = 10240
NPK = NP // 128
NPH = NP * H // 128
NB = 3

_mesh = plsc.VectorSubcoreMesh(core_axis_name="c", subcore_axis_name="s")


@functools.partial(
    pl.kernel,
    out_type=jax.ShapeDtypeStruct((NC, NP), jnp.float32),
    mesh=_mesh,
    compiler_params=pltpu.CompilerParams(use_tc_tiling_on_sc=False),
    scratch_types=[
        pltpu.VMEM((K, CHUNK), jnp.int32),
        pltpu.VMEM((CHUNK,), jnp.float32),
        pltpu.VMEM_SHARED((NP,), jnp.float32),
    ],
)
def _sc_deg(dst_hbm, ones_hbm, zeros_hbm, out_hbm, dst_v, ones_v, acc_sh):
    c = lax.axis_index("c")
    s = lax.axis_index("s")
    wid = c * NS + s
    rp = NP // NS
    pltpu.sync_copy(dst_hbm.at[wid], dst_v)
    pltpu.sync_copy(ones_hbm, ones_v)
    pltpu.sync_copy(zeros_hbm.at[pl.ds(s * rp, rp)],
                    acc_sh.at[pl.ds(s * rp, rp)])
    plsc.subcore_barrier()

    def step(j, _):
        pltpu.sync_copy(ones_v, acc_sh.at[dst_v.at[j]], add=True)
        return ()

    lax.fori_loop(0, K, step, ())
    plsc.subcore_barrier()

    pltpu.sync_copy(acc_sh.at[pl.ds(s * rp, rp)],
                    out_hbm.at[c].at[pl.ds(s * rp, rp)])


@functools.partial(
    pl.kernel,
    out_type=jax.ShapeDtypeStruct((NC, NP, H), jnp.float32),
    mesh=_mesh,
    compiler_params=pltpu.CompilerParams(use_tc_tiling_on_sc=False),
    scratch_types=[
        pltpu.VMEM((K, CHUNK), jnp.int32),
        pltpu.VMEM((K, CHUNK), jnp.int32),
        pltpu.VMEM((NB, CHUNK, H), jnp.float32),
        pltpu.VMEM_SHARED((NP, H), jnp.float32),
        pltpu.VMEM_SHARED((NP, H), jnp.float32),
        pltpu.SemaphoreType.DMA,
        pltpu.SemaphoreType.DMA,
        pltpu.SemaphoreType.DMA,
    ],
)
def _sc_agg1(src_hbm, dst_hbm, p1_hbm, zeros_hbm, out_hbm,
             src_v, dst_v, rows_v, p1_sh, acc_sh, sem0, sem1, sem2):
    c = lax.axis_index("c")
    s = lax.axis_index("s")
    wid = c * NS + s
    rp = NP // NS
    sems = [sem0, sem1, sem2]
    pltpu.sync_copy(src_hbm.at[wid], src_v)
    pltpu.sync_copy(dst_hbm.at[wid], dst_v)
    pltpu.sync_copy(p1_hbm.at[pl.ds(s * rp, rp)],
                    p1_sh.at[pl.ds(s * rp, rp)])
    pltpu.sync_copy(zeros_hbm.at[pl.ds(s * rp, rp)],
                    acc_sh.at[pl.ds(s * rp, rp)])
    plsc.subcore_barrier()

    for b in range(NB):
        pltpu.async_copy(p1_sh.at[src_v.at[b]], rows_v.at[b], sems[b])

    def outer(g, _):
        for b in range(NB):
            j = g * NB + b
            pltpu.make_async_copy(p1_sh.at[src_v.at[j]],
                                  rows_v.at[b], sems[b]).wait()
            pltpu.sync_copy(rows_v.at[b], acc_sh.at[dst_v.at[j]], add=True)
            nj = j + NB

            @pl.when(nj < K)
            def _():
                pltpu.async_copy(p1_sh.at[src_v.at[nj]], rows_v.at[b],
                                 sems[b])
        return ()

    lax.fori_loop(0, K // NB, outer, ())
    plsc.subcore_barrier()

    pltpu.sync_copy(acc_sh.at[pl.ds(s * rp, rp)],
                    out_hbm.at[c].at[pl.ds(s * rp, rp)])


@functools.partial(
    pl.kernel,
    out_type=jax.ShapeDtypeStruct((NC, NP), jnp.float32),
    mesh=_mesh,
    compiler_params=pltpu.CompilerParams(use_tc_tiling_on_sc=False),
    scratch_types=[
        pltpu.VMEM((K, CHUNK), jnp.int32),
        pltpu.VMEM((K, CHUNK), jnp.int32),
        pltpu.VMEM((NB, CHUNK), jnp.float32),
        pltpu.VMEM_SHARED((NP,), jnp.float32),
        pltpu.VMEM_SHARED((NP,), jnp.float32),
        pltpu.SemaphoreType.DMA,
        pltpu.SemaphoreType.DMA,
        pltpu.SemaphoreType.DMA,
    ],
)
def _sc_agg2(src_hbm, dst_hbm, p2_hbm, zeros_hbm, out_hbm,
             src_v, dst_v, rows_v, p2_sh, acc_sh, sem0, sem1, sem2):
    c = lax.axis_index("c")
    s = lax.axis_index("s")
    wid = c * NS + s
    rp = NP // NS
    sems = [sem0, sem1, sem2]
    pltpu.sync_copy(src_hbm.at[wid], src_v)
    pltpu.sync_copy(dst_hbm.at[wid], dst_v)
    pltpu.sync_copy(p2_hbm.at[pl.ds(s * rp, rp)],
                    p2_sh.at[pl.ds(s * rp, rp)])
    pltpu.sync_copy(zeros_hbm.at[pl.ds(s * rp, rp)],
                    acc_sh.at[pl.ds(s * rp, rp)])
    plsc.subcore_barrier()

    for b in range(NB):
        pltpu.async_copy(p2_sh.at[src_v.at[b]], rows_v.at[b], sems[b])

    def outer(g, _):
        for b in range(NB):
            j = g * NB + b
            pltpu.make_async_copy(p2_sh.at[src_v.at[j]],
                                  rows_v.at[b], sems[b]).wait()
            pltpu.sync_copy(rows_v.at[b], acc_sh.at[dst_v.at[j]], add=True)
            nj = j + NB

            @pl.when(nj < K)
            def _():
                pltpu.async_copy(p2_sh.at[src_v.at[nj]], rows_v.at[b],
                                 sems[b])
        return ()

    lax.fori_loop(0, K // NB, outer, ())
    plsc.subcore_barrier()

    pltpu.sync_copy(acc_sh.at[pl.ds(s * rp, rp)],
                    out_hbm.at[c].at[pl.ds(s * rp, rp)])


def _tc_u_body(x_ref, w1blk_ref, u_ref):
    u = jnp.dot(x_ref[pl.ds(0, NPH), :], w1blk_ref[pl.ds(0, D), :],
                preferred_element_type=jnp.float32)
    for a in range(1, 4):
        u += jnp.dot(x_ref[pl.ds(a * NPH, NPH), :],
                     w1blk_ref[pl.ds(a * D, D), :],
                     preferred_element_type=jnp.float32)
    u_ref[...] = u


def _tc_a_body(degexp_ref, degpk_ref, u_ref, p1pk_ref, dexp_ref, dpk_ref):
    dexp = lax.rsqrt(jnp.maximum(degexp_ref[...], 1.0))
    dexp_ref[...] = dexp
    dpk_ref[...] = lax.rsqrt(
        jnp.maximum(degpk_ref[0] + degpk_ref[1], 1.0))
    p1pk_ref[...] = u_ref[...] * dexp


def _tc_b_body(s1pk_ref, dexp_ref, b1t_ref, m_ref, p2pk_ref):
    s1 = s1pk_ref[0] + s1pk_ref[1]
    h = jnp.maximum(s1 * dexp_ref[...] + b1t_ref[...], 0.0)
    p2pk_ref[...] = jnp.dot(h, m_ref[...],
                            preferred_element_type=jnp.float32) * dexp_ref[...]


def _tc_c_body(s2pk_ref, dpk_ref, b2_ref, out_ref):
    s2 = s2pk_ref[0] + s2pk_ref[1]
    out_ref[...] = s2 * dpk_ref[...] + b2_ref[...]


def kernel(x, edge_index, W1, b1, W2, b2):
    src = edge_index[0]
    dst = edge_index[1]
    pad = EP - E - NP
    loop = jnp.arange(NP, dtype=jnp.int32)
    padv = N + (jnp.arange(pad, dtype=jnp.int32) % (NP - N))
    src_a = jnp.concatenate([src, loop, padv])
    dst_a = jnp.concatenate([dst, loop, padv])
    src_p = ((src_a % NPH) * 4 + src_a // NPH).reshape(NW, K, CHUNK)
    dst_p = ((dst_a % NPH) * 4 + dst_a // NPH).reshape(NW, K, CHUNK)
    x_p = jnp.pad(x, ((0, NP - N), (0, 0)))

    ones_c = jnp.ones((CHUNK,), jnp.float32)
    zeros_n = jnp.zeros((NP,), jnp.float32)
    zeros_nh = jnp.zeros((NP, H), jnp.float32)

    degT = _sc_deg(dst_p, ones_c, zeros_n)
    degpk = degT.reshape(NC, NPK, CHUNK)
    deg_pi = degT[0] + degT[1]
    degexp = jnp.repeat(deg_pi.reshape(NPH, 4), H, axis=1)

    w1blk = jnp.kron(jnp.eye(4, dtype=jnp.float32), W1)

    u_pk = pl.pallas_call(
        _tc_u_body,
        out_shape=jax.ShapeDtypeStruct((NPH, CHUNK), jnp.float32),
    )(x_p, w1blk)

    p1pk, dexp, d_pk = pl.pallas_call(
        _tc_a_body,
        out_shape=(jax.ShapeDtypeStruct((NPH, CHUNK), jnp.float32),
                   jax.ShapeDtypeStruct((NPH, CHUNK), jnp.float32),
                   jax.ShapeDtypeStruct((NPK, CHUNK), jnp.float32)),
    )(degexp, degpk, u_pk)

    s1p = _sc_agg1(src_p, dst_p, p1pk.reshape(NP, H), zeros_nh)
    s1pk = s1p.reshape(NC, NPH, CHUNK)

    b1t = jnp.tile(b1, 4).reshape(1, CHUNK)
    m_blk = jnp.kron(jnp.eye(4, dtype=jnp.float32),
                     jnp.tile(W2, (1, H)))

    p2pk = pl.pallas_call(
        _tc_b_body,
        out_shape=jax.ShapeDtypeStruct((NPH, CHUNK), jnp.float32),
    )(s1pk, dexp, b1t, m_blk)

    p2_flat = p2pk.reshape(NP, H)[:, 0]

    s2T = _sc_agg2(src_p, dst_p, p2_flat, zeros_n)
    s2pk = s2T.reshape(NC, NPK, CHUNK)

    outpk = pl.pallas_call(
        _tc_c_body,
        out_shape=jax.ShapeDtypeStruct((NPK, CHUNK), jnp.float32),
    )(s2pk, d_pk, b2.reshape(1, 1))

    out_n = outpk.reshape(NPH, 4).T.reshape(NP)
    return out_n[:N]


# speedup vs baseline: 78.3297x; 1.0217x over previous
import functools

import jax
import jax.numpy as jnp
from jax import lax
from jax.experimental import pallas as pl
from jax.experimental.pallas import tpu as pltpu
from jax.experimental.pallas import tpu_sc as plsc

NC = 2
NS = 16
NW = NC * NS

N = 10000
E = 320000
D = 128
H = 32

CHUNK = 128
K = 81
T = K * CHUNK
EP = NW * T
NP = 10240
NPK = NP // 128
NPH = NP * H // 128
NB = 3

_mesh = plsc.VectorSubcoreMesh(core_axis_name="c", subcore_axis_name="s")


@functools.partial(
    pl.kernel,
    out_type=jax.ShapeDtypeStruct((NC, NP), jnp.float32),
    mesh=_mesh,
    compiler_params=pltpu.CompilerParams(use_tc_tiling_on_sc=False),
    scratch_types=[
        pltpu.VMEM((K, CHUNK), jnp.int32),
        pltpu.VMEM((CHUNK,), jnp.float32),
        pltpu.VMEM_SHARED((NP,), jnp.float32),
    ],
)
def _sc_deg(dst_hbm, ones_hbm, zeros_hbm, out_hbm, dst_v, ones_v, acc_sh):
    c = lax.axis_index("c")
    s = lax.axis_index("s")
    wid = c * NS + s
    rp = NP // NS
    pltpu.sync_copy(dst_hbm.at[wid], dst_v)
    pltpu.sync_copy(ones_hbm, ones_v)
    pltpu.sync_copy(zeros_hbm.at[pl.ds(s * rp, rp)],
                    acc_sh.at[pl.ds(s * rp, rp)])
    plsc.subcore_barrier()

    def step(j, _):
        pltpu.sync_copy(ones_v, acc_sh.at[dst_v.at[j]], add=True)
        return ()

    lax.fori_loop(0, K, step, ())
    plsc.subcore_barrier()

    pltpu.sync_copy(acc_sh.at[pl.ds(s * rp, rp)],
                    out_hbm.at[c].at[pl.ds(s * rp, rp)])


@functools.partial(
    pl.kernel,
    out_type=jax.ShapeDtypeStruct((NC, NP, H), jnp.float32),
    mesh=_mesh,
    compiler_params=pltpu.CompilerParams(use_tc_tiling_on_sc=False),
    scratch_types=[
        pltpu.VMEM((K, CHUNK), jnp.int32),
        pltpu.VMEM((K, CHUNK), jnp.int32),
        pltpu.VMEM((NB, CHUNK, H), jnp.float32),
        pltpu.VMEM_SHARED((NP, H), jnp.float32),
        pltpu.VMEM_SHARED((NP, H), jnp.float32),
        pltpu.SemaphoreType.DMA,
        pltpu.SemaphoreType.DMA,
        pltpu.SemaphoreType.DMA,
    ],
)
def _sc_agg1(src_hbm, dst_hbm, p1_hbm, zeros_hbm, out_hbm,
             src_v, dst_v, rows_v, p1_sh, acc_sh, sem0, sem1, sem2):
    c = lax.axis_index("c")
    s = lax.axis_index("s")
    wid = c * NS + s
    rp = NP // NS
    sems = [sem0, sem1, sem2]
    pltpu.sync_copy(src_hbm.at[wid], src_v)
    pltpu.sync_copy(dst_hbm.at[wid], dst_v)
    pltpu.sync_copy(p1_hbm.at[pl.ds(s * rp, rp)],
                    p1_sh.at[pl.ds(s * rp, rp)])
    pltpu.sync_copy(zeros_hbm.at[pl.ds(s * rp, rp)],
                    acc_sh.at[pl.ds(s * rp, rp)])
    plsc.subcore_barrier()

    for b in range(NB):
        pltpu.async_copy(p1_sh.at[src_v.at[b]], rows_v.at[b], sems[b])

    def outer(g, _):
        for b in range(NB):
            j = g * NB + b
            pltpu.make_async_copy(p1_sh.at[src_v.at[j]],
                                  rows_v.at[b], sems[b]).wait()
            pltpu.sync_copy(rows_v.at[b], acc_sh.at[dst_v.at[j]], add=True)
            nj = j + NB

            @pl.when(nj < K)
            def _():
                pltpu.async_copy(p1_sh.at[src_v.at[nj]], rows_v.at[b],
                                 sems[b])
        return ()

    lax.fori_loop(0, K // NB, outer, ())
    plsc.subcore_barrier()

    pltpu.sync_copy(acc_sh.at[pl.ds(s * rp, rp)],
                    out_hbm.at[c].at[pl.ds(s * rp, rp)])


@functools.partial(
    pl.kernel,
    out_type=jax.ShapeDtypeStruct((NC, NP), jnp.float32),
    mesh=_mesh,
    compiler_params=pltpu.CompilerParams(use_tc_tiling_on_sc=False),
    scratch_types=[
        pltpu.VMEM((K, CHUNK), jnp.int32),
        pltpu.VMEM((K, CHUNK), jnp.int32),
        pltpu.VMEM((NB, CHUNK), jnp.float32),
        pltpu.VMEM_SHARED((NP,), jnp.float32),
        pltpu.VMEM_SHARED((NP,), jnp.float32),
        pltpu.SemaphoreType.DMA,
        pltpu.SemaphoreType.DMA,
        pltpu.SemaphoreType.DMA,
    ],
)
def _sc_agg2(src_hbm, dst_hbm, p2_hbm, zeros_hbm, out_hbm,
             src_v, dst_v, rows_v, p2_sh, acc_sh, sem0, sem1, sem2):
    c = lax.axis_index("c")
    s = lax.axis_index("s")
    wid = c * NS + s
    rp = NP // NS
    sems = [sem0, sem1, sem2]
    pltpu.sync_copy(src_hbm.at[wid], src_v)
    pltpu.sync_copy(dst_hbm.at[wid], dst_v)
    pltpu.sync_copy(p2_hbm.at[pl.ds(s * rp, rp)],
                    p2_sh.at[pl.ds(s * rp, rp)])
    pltpu.sync_copy(zeros_hbm.at[pl.ds(s * rp, rp)],
                    acc_sh.at[pl.ds(s * rp, rp)])
    plsc.subcore_barrier()

    for b in range(NB):
        pltpu.async_copy(p2_sh.at[src_v.at[b]], rows_v.at[b], sems[b])

    def outer(g, _):
        for b in range(NB):
            j = g * NB + b
            pltpu.make_async_copy(p2_sh.at[src_v.at[j]],
                                  rows_v.at[b], sems[b]).wait()
            pltpu.sync_copy(rows_v.at[b], acc_sh.at[dst_v.at[j]], add=True)
            nj = j + NB

            @pl.when(nj < K)
            def _():
                pltpu.async_copy(p2_sh.at[src_v.at[nj]], rows_v.at[b],
                                 sems[b])
        return ()

    lax.fori_loop(0, K // NB, outer, ())
    plsc.subcore_barrier()

    pltpu.sync_copy(acc_sh.at[pl.ds(s * rp, rp)],
                    out_hbm.at[c].at[pl.ds(s * rp, rp)])


def _tc_u_body(x_ref, w1blk_ref, u_ref):
    u = jnp.dot(x_ref[pl.ds(0, NPH), :], w1blk_ref[pl.ds(0, D), :],
                preferred_element_type=jnp.float32)
    for a in range(1, 4):
        u += jnp.dot(x_ref[pl.ds(a * NPH, NPH), :],
                     w1blk_ref[pl.ds(a * D, D), :],
                     preferred_element_type=jnp.float32)
    u_ref[...] = u


def _tc_a_body(degexp_ref, degpk_ref, u_ref, p1pk_ref, dexp_ref, dpk_ref):
    dexp = lax.rsqrt(jnp.maximum(degexp_ref[...], 1.0))
    dexp_ref[...] = dexp
    dpk_ref[...] = lax.rsqrt(
        jnp.maximum(degpk_ref[0] + degpk_ref[1], 1.0))
    p1pk_ref[...] = u_ref[...] * dexp


def _tc_b_body(s1pk_ref, dexp_ref, b1t_ref, m_ref, p2pk_ref):
    s1 = s1pk_ref[0] + s1pk_ref[1]
    h = jnp.maximum(s1 * dexp_ref[...] + b1t_ref[...], 0.0)
    p2pk_ref[...] = jnp.dot(h, m_ref[...],
                            preferred_element_type=jnp.float32) * dexp_ref[...]


def _tc_c_body(s2pk_ref, dpk_ref, b2_ref, out_ref):
    s2 = s2pk_ref[0] + s2pk_ref[1]
    out_ref[...] = s2 * dpk_ref[...] + b2_ref[...]


def kernel(x, edge_index, W1, b1, W2, b2):
    src = edge_index[0]
    dst = edge_index[1]
    pad = EP - E - NP
    loop = jnp.arange(NP, dtype=jnp.int32)
    padv = N + (jnp.arange(pad, dtype=jnp.int32) % (NP - N))
    src_a = jnp.concatenate([src, loop, padv])
    dst_a = jnp.concatenate([dst, loop, padv])

    def _pi(v):
        blk = ((v >= NPH).astype(jnp.int32)
               + (v >= 2 * NPH).astype(jnp.int32)
               + (v >= 3 * NPH).astype(jnp.int32))
        return v * 4 - blk * (4 * NPH - 1)

    src_p = _pi(src_a).reshape(NW, K, CHUNK)
    dst_p = _pi(dst_a).reshape(NW, K, CHUNK)
    x_p = jnp.pad(x, ((0, NP - N), (0, 0)))

    ones_c = jnp.ones((CHUNK,), jnp.float32)
    zeros_n = jnp.zeros((NP,), jnp.float32)
    zeros_nh = jnp.zeros((NP, H), jnp.float32)

    degT = _sc_deg(dst_p, ones_c, zeros_n)
    degpk = degT.reshape(NC, NPK, CHUNK)
    deg_pi = degT[0] + degT[1]
    degexp = jnp.repeat(deg_pi.reshape(NPH, 4), H, axis=1)

    w1blk = jnp.kron(jnp.eye(4, dtype=jnp.float32), W1)

    u_pk = pl.pallas_call(
        _tc_u_body,
        out_shape=jax.ShapeDtypeStruct((NPH, CHUNK), jnp.float32),
    )(x_p, w1blk)

    p1pk, dexp, d_pk = pl.pallas_call(
        _tc_a_body,
        out_shape=(jax.ShapeDtypeStruct((NPH, CHUNK), jnp.float32),
                   jax.ShapeDtypeStruct((NPH, CHUNK), jnp.float32),
                   jax.ShapeDtypeStruct((NPK, CHUNK), jnp.float32)),
    )(degexp, degpk, u_pk)

    s1p = _sc_agg1(src_p, dst_p, p1pk.reshape(NP, H), zeros_nh)
    s1pk = s1p.reshape(NC, NPH, CHUNK)

    b1t = jnp.tile(b1, 4).reshape(1, CHUNK)
    m_blk = jnp.kron(jnp.eye(4, dtype=jnp.float32),
                     jnp.tile(W2, (1, H)))

    p2pk = pl.pallas_call(
        _tc_b_body,
        out_shape=jax.ShapeDtypeStruct((NPH, CHUNK), jnp.float32),
    )(s1pk, dexp, b1t, m_blk)

    p2_flat = p2pk.reshape(NP, H)[:, 0]

    s2T = _sc_agg2(src_p, dst_p, p2_flat, zeros_n)
    s2pk = s2T.reshape(NC, NPK, CHUNK)

    outpk = pl.pallas_call(
        _tc_c_body,
        out_shape=jax.ShapeDtypeStruct((NPK, CHUNK), jnp.float32),
    )(s2pk, d_pk, b2.reshape(1, 1))

    out_n = outpk.reshape(NPH, 4).T.reshape(NP)
    return out_n[:N]
